# Initial kernel scaffold; baseline (speedup 1.0000x reference)
import jax
import jax.numpy as jnp
from jax.experimental import pallas as pl


def kernel(x, train_pos_edge_index, edge_type, pos_edge_index, neg_edge_index, basis1, comp1, root1, bias1, basis2, comp2, root2, bias2, fc1_W, fc1_b):
    raise NotImplementedError("write your pallas kernel here")


# speedup vs baseline: 17.3698x; 17.3698x over previous
import jax
import jax.numpy as jnp
from jax import lax
from jax.experimental import pallas as pl
from jax.experimental.pallas import tpu as pltpu
from jax.experimental.pallas import tpu_sc as plsc

_N = 10000
_E = 320000
_D = 128
_H = 16
_R = 8
_NB = 30
_BN = 1000
_GRID = _N // _BN

_NC, _NS = 2, 16
_NW = _NC * _NS
_EPT = _E // _NW
_CH = 80
_NCHUNK = _EPT // _CH
_SROWS = _N * _R
_SPT = _SROWS // _NS


def _wall_from_bases(comp_ref, basis_ref, wall_ref, din):
    for r in range(_R):
        acc = jnp.zeros((din, _H), jnp.float32)
        for b in range(_NB):
            acc = acc + comp_ref[r, b] * basis_ref[b]
        wall_ref[:, r * _H:(r + 1) * _H] = acc


def _tc_prep_body(comp_ref, basis_ref, x_ref, root_ref, bias_ref,
                  z_ref, r_ref, wall_ref):
    @pl.when(pl.program_id(0) == 0)
    def _():
        _wall_from_bases(comp_ref, basis_ref, wall_ref, _D)

    xb = x_ref[...]
    z_ref[...] = jnp.dot(xb, wall_ref[...], preferred_element_type=jnp.float32)
    r_ref[...] = (jnp.dot(xb, root_ref[...], preferred_element_type=jnp.float32)
                  + bias_ref[...])


_tc_prep = pl.pallas_call(
    _tc_prep_body,
    grid=(_GRID,),
    in_specs=[
        pl.BlockSpec(memory_space=pltpu.SMEM),
        pl.BlockSpec((_NB, _D, _H), lambda i: (0, 0, 0)),
        pl.BlockSpec((_BN, _D), lambda i: (i, 0)),
        pl.BlockSpec((_D, _H), lambda i: (0, 0)),
        pl.BlockSpec((1, _H), lambda i: (0, 0)),
    ],
    out_specs=[
        pl.BlockSpec((_BN, _R * _H), lambda i: (i, 0)),
        pl.BlockSpec((_BN, _H), lambda i: (i, 0)),
    ],
    out_shape=[
        jax.ShapeDtypeStruct((_N, _R * _H), jnp.float32),
        jax.ShapeDtypeStruct((_N, _H), jnp.float32),
    ],
    scratch_shapes=[pltpu.VMEM((_D, _R * _H), jnp.float32)],
)


def _normalized_msg(s_ref, invf):
    prod = (s_ref[0] + s_ref[1]) * invf
    msg = jnp.zeros((_BN, _H), jnp.float32)
    for r in range(_R):
        msg = msg + prod[:, r * _H:(r + 1) * _H]
    return msg


def _tc_mid_body(comp_ref, basis_ref, s_ref, cnt_ref, r1_ref, root_ref,
                 bias_ref, z2_ref, r2_ref, inv_ref, wall_ref):
    @pl.when(pl.program_id(0) == 0)
    def _():
        _wall_from_bases(comp_ref, basis_ref, wall_ref, _H)

    cntf = cnt_ref[0] + cnt_ref[1]
    invf = 1.0 / jnp.maximum(cntf, 1.0)
    inv_ref[...] = invf
    h1 = jax.nn.relu(r1_ref[...] + _normalized_msg(s_ref, invf))
    z2_ref[...] = jnp.dot(h1, wall_ref[...], preferred_element_type=jnp.float32)
    r2_ref[...] = (jnp.dot(h1, root_ref[...], preferred_element_type=jnp.float32)
                   + bias_ref[...])


_tc_mid = pl.pallas_call(
    _tc_mid_body,
    grid=(_GRID,),
    in_specs=[
        pl.BlockSpec(memory_space=pltpu.SMEM),
        pl.BlockSpec((_NB, _H, _H), lambda i: (0, 0, 0)),
        pl.BlockSpec((2, _BN, _R * _H), lambda i: (0, i, 0)),
        pl.BlockSpec((2, _BN, _R * _H), lambda i: (0, i, 0)),
        pl.BlockSpec((_BN, _H), lambda i: (i, 0)),
        pl.BlockSpec((_H, _H), lambda i: (0, 0)),
        pl.BlockSpec((1, _H), lambda i: (0, 0)),
    ],
    out_specs=[
        pl.BlockSpec((_BN, _R * _H), lambda i: (i, 0)),
        pl.BlockSpec((_BN, _H), lambda i: (i, 0)),
        pl.BlockSpec((_BN, _R * _H), lambda i: (i, 0)),
    ],
    out_shape=[
        jax.ShapeDtypeStruct((_N, _R * _H), jnp.float32),
        jax.ShapeDtypeStruct((_N, _H), jnp.float32),
        jax.ShapeDtypeStruct((_N, _R * _H), jnp.float32),
    ],
    scratch_shapes=[pltpu.VMEM((_H, _R * _H), jnp.float32)],
)


def _tc_final_body(s_ref, inv_ref, r2_ref, fwt_ref, fb_ref, out_ref):
    y2 = r2_ref[...] + _normalized_msg(s_ref, inv_ref[...])
    out_ref[...] = (jnp.dot(y2, fwt_ref[...], preferred_element_type=jnp.float32)
                    + fb_ref[...])


_tc_final = pl.pallas_call(
    _tc_final_body,
    grid=(_GRID,),
    in_specs=[
        pl.BlockSpec((2, _BN, _R * _H), lambda i: (0, i, 0)),
        pl.BlockSpec((_BN, _R * _H), lambda i: (i, 0)),
        pl.BlockSpec((_BN, _H), lambda i: (i, 0)),
        pl.BlockSpec((_H, _H), lambda i: (0, 0)),
        pl.BlockSpec((1, _H), lambda i: (0, 0)),
    ],
    out_specs=[pl.BlockSpec((_BN, _H), lambda i: (i, 0))],
    out_shape=[jax.ShapeDtypeStruct((_N, _H), jnp.float32)],
)


_MESH = plsc.VectorSubcoreMesh(core_axis_name="c", subcore_axis_name="s",
                               num_cores=_NC, num_subcores=_NS)


def _sc_cnt_body(dstv, tpv, zeros_hbm, cnt_out, dbuf, tbuf, sidx, ones, c_sh):
    c = lax.axis_index("c")
    s = lax.axis_index("s")
    wid = s * _NC + c

    pltpu.sync_copy(zeros_hbm.at[pl.ds(s * _SPT, _SPT)],
                    c_sh.at[pl.ds(s * _SPT, _SPT)])
    one_row = jnp.ones((_H,), jnp.float32)
    for q in range(_CH):
        ones[q] = one_row
    plsc.subcore_barrier()

    base = wid * _EPT

    def chunk(j, carry):
        off = base + j * _CH
        pltpu.sync_copy(dstv.at[pl.ds(off, _CH)], dbuf)
        pltpu.sync_copy(tpv.at[pl.ds(off, _CH)], tbuf)
        for k in range(_CH // 16):
            sl = pl.ds(k * 16, 16)
            sidx[0, sl] = dbuf[sl] * _R + tbuf[sl]
        pltpu.sync_copy(ones, c_sh.at[sidx.at[0]], add=True)
        return carry

    lax.fori_loop(0, _NCHUNK, chunk, 0)
    plsc.subcore_barrier()
    pltpu.sync_copy(c_sh.at[pl.ds(s * _SPT, _SPT)],
                    cnt_out.at[c].at[pl.ds(s * _SPT, _SPT)])


_sc_cnt = pl.kernel(
    _sc_cnt_body,
    out_type=[jax.ShapeDtypeStruct((_NC, _SROWS, _H), jnp.float32)],
    mesh=_MESH,
    compiler_params=pltpu.CompilerParams(use_tc_tiling_on_sc=False, needs_layout_passes=False),
    scratch_types=[
        pltpu.VMEM((_CH,), jnp.int32),
        pltpu.VMEM((_CH,), jnp.int32),
        pltpu.VMEM((1, _CH), jnp.int32),
        pltpu.VMEM((_CH, _H), jnp.float32),
        pltpu.VMEM_SHARED((_SROWS, _H), jnp.float32),
    ],
)


def _sc_msg_body(zflat, srcv, dstv, tpv, zeros_hbm,
                 s_out, sbuf, dbuf, tbuf, gidx, sidx, rows, s_sh, sem):
    c = lax.axis_index("c")
    s = lax.axis_index("s")
    wid = s * _NC + c

    pltpu.sync_copy(zeros_hbm.at[pl.ds(s * _SPT, _SPT)],
                    s_sh.at[pl.ds(s * _SPT, _SPT)])
    plsc.subcore_barrier()

    base = wid * _EPT

    def chunk(j, carry):
        off = base + j * _CH
        pltpu.sync_copy(srcv.at[pl.ds(off, _CH)], sbuf)
        pltpu.sync_copy(dstv.at[pl.ds(off, _CH)], dbuf)
        pltpu.sync_copy(tpv.at[pl.ds(off, _CH)], tbuf)
        for k in range(_CH // 16):
            sl = pl.ds(k * 16, 16)
            tv = tbuf[sl]
            gidx[sl] = sbuf[sl] * _R + tv
            sidx[0, sl] = dbuf[sl] * _R + tv
        pltpu.async_copy(zflat.at[gidx], rows, sem).wait()
        pltpu.sync_copy(rows, s_sh.at[sidx.at[0]], add=True)
        return carry

    lax.fori_loop(0, _NCHUNK, chunk, 0)
    plsc.subcore_barrier()

    pltpu.sync_copy(s_sh.at[pl.ds(s * _SPT, _SPT)],
                    s_out.at[c].at[pl.ds(s * _SPT, _SPT)])


_sc_msg = pl.kernel(
    _sc_msg_body,
    out_type=[jax.ShapeDtypeStruct((_NC, _SROWS, _H), jnp.float32)],
    mesh=_MESH,
    compiler_params=pltpu.CompilerParams(use_tc_tiling_on_sc=False, needs_layout_passes=False),
    scratch_types=[
        pltpu.VMEM((_CH,), jnp.int32),
        pltpu.VMEM((_CH,), jnp.int32),
        pltpu.VMEM((_CH,), jnp.int32),
        pltpu.VMEM((_CH,), jnp.int32),
        pltpu.VMEM((1, _CH), jnp.int32),
        pltpu.VMEM((_CH, _H), jnp.float32),
        pltpu.VMEM_SHARED((_SROWS, _H), jnp.float32),
        pltpu.SemaphoreType.DMA,
    ],
)


def _sc_scores_body(outn, srcd, dstd, sc_out,
                    sbuf, dbuf, rows_i, rows_j, scbuf, sem, sem2):
    c = lax.axis_index("c")
    s = lax.axis_index("s")
    wid = s * _NC + c
    base = wid * _EPT

    def chunk(j, carry):
        off = base + j * _CH
        pltpu.sync_copy(srcd.at[pl.ds(off, _CH)], sbuf)
        pltpu.sync_copy(dstd.at[pl.ds(off, _CH)], dbuf)
        cp1 = pltpu.async_copy(outn.at[sbuf], rows_i, sem)
        cp2 = pltpu.async_copy(outn.at[dbuf], rows_j, sem2)
        cp1.wait()
        cp2.wait()

        ii = lax.iota(jnp.int32, 16)
        for g in range(_CH // 16):
            acc = jnp.zeros((16,), jnp.float32)
            for u in range(16):
                e = g * 16 + u
                p = rows_i[e] * rows_j[e]
                acc = acc + jnp.where(ii == u, jnp.sum(p), 0.0)
            scbuf[pl.ds(g * 16, 16)] = acc
        pltpu.sync_copy(scbuf, sc_out.at[pl.ds(off, _CH)])
        return carry

    lax.fori_loop(0, _NCHUNK, chunk, 0)


_sc_scores = pl.kernel(
    _sc_scores_body,
    out_type=[jax.ShapeDtypeStruct((_E,), jnp.float32)],
    mesh=_MESH,
    compiler_params=pltpu.CompilerParams(use_tc_tiling_on_sc=False, needs_layout_passes=False),
    scratch_types=[
        pltpu.VMEM((_CH,), jnp.int32),
        pltpu.VMEM((_CH,), jnp.int32),
        pltpu.VMEM((_CH, _H), jnp.float32),
        pltpu.VMEM((_CH, _H), jnp.float32),
        pltpu.VMEM((_CH,), jnp.float32),
        pltpu.SemaphoreType.DMA,
        pltpu.SemaphoreType.DMA,
    ],
)


def kernel(x, train_pos_edge_index, edge_type, pos_edge_index, neg_edge_index,
           basis1, comp1, root1, bias1, basis2, comp2, root2, bias2,
           fc1_W, fc1_b):
    src = train_pos_edge_index[0]
    dst = train_pos_edge_index[1]
    et = edge_type

    zeros_hbm = jnp.zeros((_SROWS, _H), jnp.float32)
    (cnt,) = _sc_cnt(dst, et, zeros_hbm)

    z1, r1 = _tc_prep(comp1, basis1, x, root1, bias1.reshape(1, _H))
    (s1,) = _sc_msg(z1.reshape(_SROWS, _H), src, dst, et, zeros_hbm)

    z2, r2, inv = _tc_mid(comp2, basis2, s1.reshape(_NC, _N, _R * _H),
                          cnt.reshape(_NC, _N, _R * _H), r1, root2,
                          bias2.reshape(1, _H))

    (s2,) = _sc_msg(z2.reshape(_SROWS, _H), src, dst, et, zeros_hbm)

    (out,) = _tc_final(s2.reshape(_NC, _N, _R * _H), inv, r2,
                       fc1_W.T, fc1_b.reshape(1, _H))

    tot_src = jnp.concatenate([pos_edge_index[0], neg_edge_index[0]])
    tot_dst = jnp.concatenate([pos_edge_index[1], neg_edge_index[1]])
    (scores,) = _sc_scores(out, tot_src, tot_dst)
    return scores, out


# speedup vs baseline: 66.3769x; 3.8214x over previous
import jax
import jax.numpy as jnp
from jax import lax
from jax.experimental import pallas as pl
from jax.experimental.pallas import tpu as pltpu
from jax.experimental.pallas import tpu_sc as plsc

_N = 10000
_E = 320000
_D = 128
_H = 16
_R = 8
_NB = 30
_BN = 1000
_GRID = _N // _BN

_NC, _NS = 2, 16
_NW = _NC * _NS
_EPT = _E // _NW
_CH = 80
_NCHUNK = _EPT // _CH
_SROWS = _N * _R
_SPT = _SROWS // _NS
_PIPE = 5


def _wall_from_bases(comp_ref, basis_ref, wall_ref, din):
    for r in range(_R):
        acc = jnp.zeros((din, _H), jnp.float32)
        for b in range(_NB):
            acc = acc + comp_ref[r, b] * basis_ref[b]
        wall_ref[:, r * _H:(r + 1) * _H] = acc


def _tc_prep_body(comp_ref, basis_ref, x_ref, root_ref, bias_ref,
                  z_ref, r_ref, wall_ref):
    @pl.when(pl.program_id(0) == 0)
    def _():
        _wall_from_bases(comp_ref, basis_ref, wall_ref, _D)

    xb = x_ref[...]
    z_ref[...] = jnp.dot(xb, wall_ref[...], preferred_element_type=jnp.float32)
    r_ref[...] = (jnp.dot(xb, root_ref[...], preferred_element_type=jnp.float32)
                  + bias_ref[...])


_tc_prep = pl.pallas_call(
    _tc_prep_body,
    grid=(_GRID,),
    in_specs=[
        pl.BlockSpec(memory_space=pltpu.SMEM),
        pl.BlockSpec((_NB, _D, _H), lambda i: (0, 0, 0)),
        pl.BlockSpec((_BN, _D), lambda i: (i, 0)),
        pl.BlockSpec((_D, _H), lambda i: (0, 0)),
        pl.BlockSpec((1, _H), lambda i: (0, 0)),
    ],
    out_specs=[
        pl.BlockSpec((_BN, _R * _H), lambda i: (i, 0)),
        pl.BlockSpec((_BN, _H), lambda i: (i, 0)),
    ],
    out_shape=[
        jax.ShapeDtypeStruct((_N, _R * _H), jnp.float32),
        jax.ShapeDtypeStruct((_N, _H), jnp.float32),
    ],
    scratch_shapes=[pltpu.VMEM((_D, _R * _H), jnp.float32)],
)


def _normalized_msg(s_ref, invf):
    prod = (s_ref[0] + s_ref[1]) * invf
    msg = jnp.zeros((_BN, _H), jnp.float32)
    for r in range(_R):
        msg = msg + prod[:, r * _H:(r + 1) * _H]
    return msg


def _tc_mid_body(comp_ref, basis_ref, s_ref, cnt_ref, r1_ref, root_ref,
                 bias_ref, z2_ref, r2_ref, inv_ref, wall_ref):
    @pl.when(pl.program_id(0) == 0)
    def _():
        _wall_from_bases(comp_ref, basis_ref, wall_ref, _H)

    cntf = cnt_ref[0] + cnt_ref[1]
    invf = 1.0 / jnp.maximum(cntf, 1.0)
    inv_ref[...] = invf
    h1 = jax.nn.relu(r1_ref[...] + _normalized_msg(s_ref, invf))
    z2_ref[...] = jnp.dot(h1, wall_ref[...], preferred_element_type=jnp.float32)
    r2_ref[...] = (jnp.dot(h1, root_ref[...], preferred_element_type=jnp.float32)
                   + bias_ref[...])


_tc_mid = pl.pallas_call(
    _tc_mid_body,
    grid=(_GRID,),
    in_specs=[
        pl.BlockSpec(memory_space=pltpu.SMEM),
        pl.BlockSpec((_NB, _H, _H), lambda i: (0, 0, 0)),
        pl.BlockSpec((2, _BN, _R * _H), lambda i: (0, i, 0)),
        pl.BlockSpec((2, _BN, _R * _H), lambda i: (0, i, 0)),
        pl.BlockSpec((_BN, _H), lambda i: (i, 0)),
        pl.BlockSpec((_H, _H), lambda i: (0, 0)),
        pl.BlockSpec((1, _H), lambda i: (0, 0)),
    ],
    out_specs=[
        pl.BlockSpec((_BN, _R * _H), lambda i: (i, 0)),
        pl.BlockSpec((_BN, _H), lambda i: (i, 0)),
        pl.BlockSpec((_BN, _R * _H), lambda i: (i, 0)),
    ],
    out_shape=[
        jax.ShapeDtypeStruct((_N, _R * _H), jnp.float32),
        jax.ShapeDtypeStruct((_N, _H), jnp.float32),
        jax.ShapeDtypeStruct((_N, _R * _H), jnp.float32),
    ],
    scratch_shapes=[pltpu.VMEM((_H, _R * _H), jnp.float32)],
)


def _tc_final_body(s_ref, inv_ref, r2_ref, fwt_ref, fb_ref, out_ref):
    y2 = r2_ref[...] + _normalized_msg(s_ref, inv_ref[...])
    out_ref[...] = (jnp.dot(y2, fwt_ref[...], preferred_element_type=jnp.float32)
                    + fb_ref[...])


_tc_final = pl.pallas_call(
    _tc_final_body,
    grid=(_GRID,),
    in_specs=[
        pl.BlockSpec((2, _BN, _R * _H), lambda i: (0, i, 0)),
        pl.BlockSpec((_BN, _R * _H), lambda i: (i, 0)),
        pl.BlockSpec((_BN, _H), lambda i: (i, 0)),
        pl.BlockSpec((_H, _H), lambda i: (0, 0)),
        pl.BlockSpec((1, _H), lambda i: (0, 0)),
    ],
    out_specs=[pl.BlockSpec((_BN, _H), lambda i: (i, 0))],
    out_shape=[jax.ShapeDtypeStruct((_N, _H), jnp.float32)],
)


_MESH = plsc.VectorSubcoreMesh(core_axis_name="c", subcore_axis_name="s",
                               num_cores=_NC, num_subcores=_NS)


def _sc_cnt_body(dstv, tpv, zeros_hbm, cnt_out, dall, tall, sidx, ones, c_sh,
                 sg0, sg1, sg2, sg3, sg4):
    c = lax.axis_index("c")
    s = lax.axis_index("s")
    wid = s * _NC + c

    pltpu.sync_copy(zeros_hbm.at[pl.ds(s * _SPT, _SPT)],
                    c_sh.at[pl.ds(s * _SPT, _SPT)])
    one_row = jnp.ones((_H,), jnp.float32)
    for q in range(_CH):
        ones[q] = one_row
    base = wid * _EPT
    pltpu.sync_copy(dstv.at[pl.ds(base, _EPT)], dall)
    pltpu.sync_copy(tpv.at[pl.ds(base, _EPT)], tall)

    def pre(j, carry):
        for k in range(_CH // 16):
            sl = pl.ds(j * _CH + k * 16, 16)
            sidx[j, pl.ds(k * 16, 16)] = dall[sl] * _R + tall[sl]
        return carry

    lax.fori_loop(0, _NCHUNK, pre, 0)
    plsc.subcore_barrier()

    sems = [sg0, sg1, sg2, sg3, sg4]

    def group(g, carry):
        cps = []
        for b in range(_PIPE):
            j = g * _PIPE + b
            cps.append(pltpu.async_copy(
                ones, c_sh.at[sidx.at[j]], sems[b], add=True))
        for cp in cps:
            cp.wait()
        return carry

    lax.fori_loop(0, _NCHUNK // _PIPE, group, 0)
    plsc.subcore_barrier()
    pltpu.sync_copy(c_sh.at[pl.ds(s * _SPT, _SPT)],
                    cnt_out.at[c].at[pl.ds(s * _SPT, _SPT)])


_sc_cnt = pl.kernel(
    _sc_cnt_body,
    out_type=[jax.ShapeDtypeStruct((_NC, _SROWS, _H), jnp.float32)],
    mesh=_MESH,
    compiler_params=pltpu.CompilerParams(use_tc_tiling_on_sc=False, needs_layout_passes=False),
    scratch_types=[
        pltpu.VMEM((_EPT,), jnp.int32),
        pltpu.VMEM((_EPT,), jnp.int32),
        pltpu.VMEM((_NCHUNK, _CH), jnp.int32),
        pltpu.VMEM((_CH, _H), jnp.float32),
        pltpu.VMEM_SHARED((_SROWS, _H), jnp.float32),
    ] + [pltpu.SemaphoreType.DMA] * _PIPE,
)


def _sc_msg_body(zflat, srcv, dstv, tpv, zeros_hbm,
                 s_out, sall, dall, tall, sidx, rows, s_sh,
                 sg0, sg1, sg2, sg3, sg4, ss0, ss1, ss2, ss3, ss4):
    c = lax.axis_index("c")
    s = lax.axis_index("s")
    wid = s * _NC + c

    pltpu.sync_copy(zeros_hbm.at[pl.ds(s * _SPT, _SPT)],
                    s_sh.at[pl.ds(s * _SPT, _SPT)])
    base = wid * _EPT
    pltpu.sync_copy(srcv.at[pl.ds(base, _EPT)], sall)
    pltpu.sync_copy(dstv.at[pl.ds(base, _EPT)], dall)
    pltpu.sync_copy(tpv.at[pl.ds(base, _EPT)], tall)

    def pre(j, carry):
        for k in range(_CH // 16):
            sl = pl.ds(j * _CH + k * 16, 16)
            tv = tall[sl]
            sall[sl] = sall[sl] * _R + tv
            sidx[j, pl.ds(k * 16, 16)] = dall[sl] * _R + tv
        return carry

    lax.fori_loop(0, _NCHUNK, pre, 0)
    plsc.subcore_barrier()

    gsems = [sg0, sg1, sg2, sg3, sg4]
    ssems = [ss0, ss1, ss2, ss3, ss4]

    def group(g, carry):
        gcps = []
        for b in range(_PIPE):
            j = g * _PIPE + b
            gcps.append(pltpu.async_copy(
                zflat.at[sall.at[pl.ds(j * _CH, _CH)]],
                rows.at[pl.ds(b * _CH, _CH)], gsems[b]))
        scps = []
        for b in range(_PIPE):
            j = g * _PIPE + b
            gcps[b].wait()
            scps.append(pltpu.async_copy(
                rows.at[pl.ds(b * _CH, _CH)],
                s_sh.at[sidx.at[j]], ssems[b], add=True))
        for cp in scps:
            cp.wait()
        return carry

    lax.fori_loop(0, _NCHUNK // _PIPE, group, 0)
    plsc.subcore_barrier()

    pltpu.sync_copy(s_sh.at[pl.ds(s * _SPT, _SPT)],
                    s_out.at[c].at[pl.ds(s * _SPT, _SPT)])


_sc_msg = pl.kernel(
    _sc_msg_body,
    out_type=[jax.ShapeDtypeStruct((_NC, _SROWS, _H), jnp.float32)],
    mesh=_MESH,
    compiler_params=pltpu.CompilerParams(use_tc_tiling_on_sc=False, needs_layout_passes=False),
    scratch_types=[
        pltpu.VMEM((_EPT,), jnp.int32),
        pltpu.VMEM((_EPT,), jnp.int32),
        pltpu.VMEM((_EPT,), jnp.int32),
        pltpu.VMEM((_NCHUNK, _CH), jnp.int32),
        pltpu.VMEM((_PIPE * _CH, _H), jnp.float32),
        pltpu.VMEM_SHARED((_SROWS, _H), jnp.float32),
    ] + [pltpu.SemaphoreType.DMA] * (2 * _PIPE),
)


def _sc_scores_body(outn, srcd, dstd, sc_out,
                    sall, dall, rows_i, rows_j, scbuf, *sems):
    c = lax.axis_index("c")
    s = lax.axis_index("s")
    wid = s * _NC + c
    base = wid * _EPT
    pltpu.sync_copy(srcd.at[pl.ds(base, _EPT)], sall)
    pltpu.sync_copy(dstd.at[pl.ds(base, _EPT)], dall)

    isems = sems[:_PIPE]
    jsems = sems[_PIPE:]
    ii = lax.iota(jnp.int32, 16)

    def group(g, carry):
        cps = []
        for b in range(_PIPE):
            sl = pl.ds((g * _PIPE + b) * _CH, _CH)
            dsl = pl.ds(b * _CH, _CH)
            cps.append((pltpu.async_copy(outn.at[sall.at[sl]],
                                         rows_i.at[dsl], isems[b]),
                        pltpu.async_copy(outn.at[dall.at[sl]],
                                         rows_j.at[dsl], jsems[b])))
        for b in range(_PIPE):
            cpi, cpj = cps[b]
            cpi.wait()
            cpj.wait()
            for q in range(_CH // 16):
                acc = jnp.zeros((16,), jnp.float32)
                for u in range(16):
                    e = b * _CH + q * 16 + u
                    p = rows_i[e] * rows_j[e]
                    acc = acc + jnp.where(ii == u, jnp.sum(p), 0.0)
                scbuf[pl.ds(b * _CH + q * 16, 16)] = acc
        pltpu.sync_copy(scbuf, sc_out.at[pl.ds(base + g * _PIPE * _CH,
                                               _PIPE * _CH)])
        return carry

    lax.fori_loop(0, _NCHUNK // _PIPE, group, 0)


_sc_scores = pl.kernel(
    _sc_scores_body,
    out_type=[jax.ShapeDtypeStruct((_E,), jnp.float32)],
    mesh=_MESH,
    compiler_params=pltpu.CompilerParams(use_tc_tiling_on_sc=False, needs_layout_passes=False),
    scratch_types=[
        pltpu.VMEM((_EPT,), jnp.int32),
        pltpu.VMEM((_EPT,), jnp.int32),
        pltpu.VMEM((_PIPE * _CH, _H), jnp.float32),
        pltpu.VMEM((_PIPE * _CH, _H), jnp.float32),
        pltpu.VMEM((_PIPE * _CH,), jnp.float32),
    ] + [pltpu.SemaphoreType.DMA] * (2 * _PIPE),
)


def kernel(x, train_pos_edge_index, edge_type, pos_edge_index, neg_edge_index,
           basis1, comp1, root1, bias1, basis2, comp2, root2, bias2,
           fc1_W, fc1_b):
    src = train_pos_edge_index[0]
    dst = train_pos_edge_index[1]
    et = edge_type

    zeros_hbm = jnp.zeros((_SROWS, _H), jnp.float32)
    (cnt,) = _sc_cnt(dst, et, zeros_hbm)

    z1, r1 = _tc_prep(comp1, basis1, x, root1, bias1.reshape(1, _H))
    (s1,) = _sc_msg(z1.reshape(_SROWS, _H), src, dst, et, zeros_hbm)

    z2, r2, inv = _tc_mid(comp2, basis2, s1.reshape(_NC, _N, _R * _H),
                          cnt.reshape(_NC, _N, _R * _H), r1, root2,
                          bias2.reshape(1, _H))

    (s2,) = _sc_msg(z2.reshape(_SROWS, _H), src, dst, et, zeros_hbm)

    (out,) = _tc_final(s2.reshape(_NC, _N, _R * _H), inv, r2,
                       fc1_W.T, fc1_b.reshape(1, _H))

    tot_src = jnp.concatenate([pos_edge_index[0], neg_edge_index[0]])
    tot_dst = jnp.concatenate([pos_edge_index[1], neg_edge_index[1]])
    (scores,) = _sc_scores(out, tot_src, tot_dst)
    return scores, out


# speedup vs baseline: 68.9757x; 1.0392x over previous
import functools

import jax
import jax.numpy as jnp
from jax import lax
from jax.experimental import pallas as pl
from jax.experimental.pallas import tpu as pltpu
from jax.experimental.pallas import tpu_sc as plsc

_N = 10000
_E = 320000
_D = 128
_H = 16
_R = 8
_NB = 30
_BN = 1000
_GRID = _N // _BN

_NC, _NS = 2, 16
_NW = _NC * _NS
_EPT = _E // _NW
_CH = 80
_NCHUNK = _EPT // _CH
_SROWS = _N * _R
_SPT = _SROWS // _NS
_PIPE = 5


def _wall_from_bases(comp_ref, basis_ref, wall_ref, din):
    for r in range(_R):
        acc = jnp.zeros((din, _H), jnp.float32)
        for b in range(_NB):
            acc = acc + comp_ref[r, b] * basis_ref[b]
        wall_ref[:, r * _H:(r + 1) * _H] = acc


def _tc_prep_body(comp_ref, basis_ref, x_ref, root_ref, bias_ref,
                  z_ref, r_ref, wall_ref):
    @pl.when(pl.program_id(0) == 0)
    def _():
        _wall_from_bases(comp_ref, basis_ref, wall_ref, _D)

    xb = x_ref[...]
    z_ref[...] = jnp.dot(xb, wall_ref[...], preferred_element_type=jnp.float32)
    r_ref[...] = (jnp.dot(xb, root_ref[...], preferred_element_type=jnp.float32)
                  + bias_ref[...])


_tc_prep = pl.pallas_call(
    _tc_prep_body,
    grid=(_GRID,),
    in_specs=[
        pl.BlockSpec(memory_space=pltpu.SMEM),
        pl.BlockSpec((_NB, _D, _H), lambda i: (0, 0, 0)),
        pl.BlockSpec((_BN, _D), lambda i: (i, 0)),
        pl.BlockSpec((_D, _H), lambda i: (0, 0)),
        pl.BlockSpec((1, _H), lambda i: (0, 0)),
    ],
    out_specs=[
        pl.BlockSpec((_BN, _R * _H), lambda i: (i, 0)),
        pl.BlockSpec((_BN, _H), lambda i: (i, 0)),
    ],
    out_shape=[
        jax.ShapeDtypeStruct((_N, _R * _H), jnp.float32),
        jax.ShapeDtypeStruct((_N, _H), jnp.float32),
    ],
    scratch_shapes=[pltpu.VMEM((_D, _R * _H), jnp.float32)],
)


def _normalized_msg(s_ref, invf):
    prod = (s_ref[0] + s_ref[1]) * invf
    msg = jnp.zeros((_BN, _H), jnp.float32)
    for r in range(_R):
        msg = msg + prod[:, r * _H:(r + 1) * _H]
    return msg


def _tc_mid_body(comp_ref, basis_ref, s_ref, cnt_ref, r1_ref, root_ref,
                 bias_ref, z2_ref, r2_ref, inv_ref, wall_ref):
    @pl.when(pl.program_id(0) == 0)
    def _():
        _wall_from_bases(comp_ref, basis_ref, wall_ref, _H)

    cntf = cnt_ref[0] + cnt_ref[1]
    invf = 1.0 / jnp.maximum(cntf, 1.0)
    inv_ref[...] = invf
    h1 = jax.nn.relu(r1_ref[...] + _normalized_msg(s_ref, invf))
    z2_ref[...] = jnp.dot(h1, wall_ref[...], preferred_element_type=jnp.float32)
    r2_ref[...] = (jnp.dot(h1, root_ref[...], preferred_element_type=jnp.float32)
                   + bias_ref[...])


_tc_mid = pl.pallas_call(
    _tc_mid_body,
    grid=(_GRID,),
    in_specs=[
        pl.BlockSpec(memory_space=pltpu.SMEM),
        pl.BlockSpec((_NB, _H, _H), lambda i: (0, 0, 0)),
        pl.BlockSpec((2, _BN, _R * _H), lambda i: (0, i, 0)),
        pl.BlockSpec((2, _BN, _R * _H), lambda i: (0, i, 0)),
        pl.BlockSpec((_BN, _H), lambda i: (i, 0)),
        pl.BlockSpec((_H, _H), lambda i: (0, 0)),
        pl.BlockSpec((1, _H), lambda i: (0, 0)),
    ],
    out_specs=[
        pl.BlockSpec((_BN, _R * _H), lambda i: (i, 0)),
        pl.BlockSpec((_BN, _H), lambda i: (i, 0)),
        pl.BlockSpec((_BN, _R * _H), lambda i: (i, 0)),
    ],
    out_shape=[
        jax.ShapeDtypeStruct((_N, _R * _H), jnp.float32),
        jax.ShapeDtypeStruct((_N, _H), jnp.float32),
        jax.ShapeDtypeStruct((_N, _R * _H), jnp.float32),
    ],
    scratch_shapes=[pltpu.VMEM((_H, _R * _H), jnp.float32)],
)


def _tc_final_body(s_ref, inv_ref, r2_ref, fwt_ref, fb_ref, out_ref):
    y2 = r2_ref[...] + _normalized_msg(s_ref, inv_ref[...])
    out_ref[...] = (jnp.dot(y2, fwt_ref[...], preferred_element_type=jnp.float32)
                    + fb_ref[...])


_tc_final = pl.pallas_call(
    _tc_final_body,
    grid=(_GRID,),
    in_specs=[
        pl.BlockSpec((2, _BN, _R * _H), lambda i: (0, i, 0)),
        pl.BlockSpec((_BN, _R * _H), lambda i: (i, 0)),
        pl.BlockSpec((_BN, _H), lambda i: (i, 0)),
        pl.BlockSpec((_H, _H), lambda i: (0, 0)),
        pl.BlockSpec((1, _H), lambda i: (0, 0)),
    ],
    out_specs=[pl.BlockSpec((_BN, _H), lambda i: (i, 0))],
    out_shape=[jax.ShapeDtypeStruct((_N, _H), jnp.float32)],
)


_MESH = plsc.VectorSubcoreMesh(core_axis_name="c", subcore_axis_name="s",
                               num_cores=_NC, num_subcores=_NS)


def _sc_msg_body(do_counts, zflat, srcv, dstv, tpv, zeros_hbm, *refs):
    if do_counts:
        (s_out, cnt_out, sall, dall, tall, sidx, rows, ones, s_sh,
         *sems) = refs
    else:
        (s_out, sall, dall, tall, sidx, rows, s_sh, *sems) = refs
    gsems = sems[:_PIPE]
    ssems = sems[_PIPE:]

    c = lax.axis_index("c")
    s = lax.axis_index("s")
    wid = s * _NC + c

    pltpu.sync_copy(zeros_hbm.at[pl.ds(s * _SPT, _SPT)],
                    s_sh.at[pl.ds(s * _SPT, _SPT)])
    base = wid * _EPT
    pltpu.sync_copy(srcv.at[pl.ds(base, _EPT)], sall)
    pltpu.sync_copy(dstv.at[pl.ds(base, _EPT)], dall)
    pltpu.sync_copy(tpv.at[pl.ds(base, _EPT)], tall)
    if do_counts:
        one_row = jnp.ones((_H,), jnp.float32)
        for q in range(_CH):
            ones[q] = one_row

    def pre(j, carry):
        for k in range(_CH // 16):
            sl = pl.ds(j * _CH + k * 16, 16)
            tv = tall[sl]
            sall[sl] = sall[sl] * _R + tv
            sidx[j, pl.ds(k * 16, 16)] = dall[sl] * _R + tv
        return carry

    lax.fori_loop(0, _NCHUNK, pre, 0)
    plsc.subcore_barrier()

    def group(g, carry):
        gcps = []
        for b in range(_PIPE):
            j = g * _PIPE + b
            gcps.append(pltpu.async_copy(
                zflat.at[sall.at[pl.ds(j * _CH, _CH)]],
                rows.at[pl.ds(b * _CH, _CH)], gsems[b]))
        scps = []
        for b in range(_PIPE):
            j = g * _PIPE + b
            gcps[b].wait()
            scps.append(pltpu.async_copy(
                rows.at[pl.ds(b * _CH, _CH)],
                s_sh.at[sidx.at[j]], ssems[b], add=True))
        for cp in scps:
            cp.wait()
        return carry

    lax.fori_loop(0, _NCHUNK // _PIPE, group, 0)
    plsc.subcore_barrier()

    pltpu.sync_copy(s_sh.at[pl.ds(s * _SPT, _SPT)],
                    s_out.at[c].at[pl.ds(s * _SPT, _SPT)])

    if do_counts:
        pltpu.sync_copy(zeros_hbm.at[pl.ds(s * _SPT, _SPT)],
                        s_sh.at[pl.ds(s * _SPT, _SPT)])
        plsc.subcore_barrier()

        def cgroup(g, carry):
            cps = []
            for b in range(_PIPE):
                j = g * _PIPE + b
                cps.append(pltpu.async_copy(
                    ones, s_sh.at[sidx.at[j]], ssems[b], add=True))
            for cp in cps:
                cp.wait()
            return carry

        lax.fori_loop(0, _NCHUNK // _PIPE, cgroup, 0)
        plsc.subcore_barrier()
        pltpu.sync_copy(s_sh.at[pl.ds(s * _SPT, _SPT)],
                        cnt_out.at[c].at[pl.ds(s * _SPT, _SPT)])


def _make_sc_msg(do_counts):
    out_type = [jax.ShapeDtypeStruct((_NC, _SROWS, _H), jnp.float32)]
    scratch = [
        pltpu.VMEM((_EPT,), jnp.int32),
        pltpu.VMEM((_EPT,), jnp.int32),
        pltpu.VMEM((_EPT,), jnp.int32),
        pltpu.VMEM((_NCHUNK, _CH), jnp.int32),
        pltpu.VMEM((_PIPE * _CH, _H), jnp.float32),
    ]
    if do_counts:
        out_type.append(jax.ShapeDtypeStruct((_NC, _SROWS, _H), jnp.float32))
        scratch.append(pltpu.VMEM((_CH, _H), jnp.float32))
    scratch.append(pltpu.VMEM_SHARED((_SROWS, _H), jnp.float32))
    scratch += [pltpu.SemaphoreType.DMA] * (2 * _PIPE)
    return pl.kernel(
        functools.partial(_sc_msg_body, do_counts),
        out_type=out_type,
        mesh=_MESH,
        compiler_params=pltpu.CompilerParams(use_tc_tiling_on_sc=False,
                                             needs_layout_passes=False),
        scratch_types=scratch,
    )


_sc_msg_counts = _make_sc_msg(True)
_sc_msg_plain = _make_sc_msg(False)


def _sc_scores_body(outn, srcd, dstd, sc_out,
                    sall, dall, rows_i, rows_j, scbuf, *sems):
    c = lax.axis_index("c")
    s = lax.axis_index("s")
    wid = s * _NC + c
    base = wid * _EPT
    pltpu.sync_copy(srcd.at[pl.ds(base, _EPT)], sall)
    pltpu.sync_copy(dstd.at[pl.ds(base, _EPT)], dall)

    isems = sems[:_PIPE]
    jsems = sems[_PIPE:]
    ii = lax.iota(jnp.int32, 16)

    def group(g, carry):
        cps = []
        for b in range(_PIPE):
            sl = pl.ds((g * _PIPE + b) * _CH, _CH)
            dsl = pl.ds(b * _CH, _CH)
            cps.append((pltpu.async_copy(outn.at[sall.at[sl]],
                                         rows_i.at[dsl], isems[b]),
                        pltpu.async_copy(outn.at[dall.at[sl]],
                                         rows_j.at[dsl], jsems[b])))
        for b in range(_PIPE):
            cpi, cpj = cps[b]
            cpi.wait()
            cpj.wait()
            for q in range(_CH // 16):
                acc = jnp.zeros((16,), jnp.float32)
                for u in range(16):
                    e = b * _CH + q * 16 + u
                    p = rows_i[e] * rows_j[e]
                    acc = acc + jnp.where(ii == u, jnp.sum(p), 0.0)
                scbuf[pl.ds(b * _CH + q * 16, 16)] = acc
        pltpu.sync_copy(scbuf, sc_out.at[pl.ds(base + g * _PIPE * _CH,
                                               _PIPE * _CH)])
        return carry

    lax.fori_loop(0, _NCHUNK // _PIPE, group, 0)


_sc_scores = pl.kernel(
    _sc_scores_body,
    out_type=[jax.ShapeDtypeStruct((_E,), jnp.float32)],
    mesh=_MESH,
    compiler_params=pltpu.CompilerParams(use_tc_tiling_on_sc=False, needs_layout_passes=False),
    scratch_types=[
        pltpu.VMEM((_EPT,), jnp.int32),
        pltpu.VMEM((_EPT,), jnp.int32),
        pltpu.VMEM((_PIPE * _CH, _H), jnp.float32),
        pltpu.VMEM((_PIPE * _CH, _H), jnp.float32),
        pltpu.VMEM((_PIPE * _CH,), jnp.float32),
    ] + [pltpu.SemaphoreType.DMA] * (2 * _PIPE),
)


def kernel(x, train_pos_edge_index, edge_type, pos_edge_index, neg_edge_index,
           basis1, comp1, root1, bias1, basis2, comp2, root2, bias2,
           fc1_W, fc1_b):
    src = train_pos_edge_index[0]
    dst = train_pos_edge_index[1]
    et = edge_type

    zeros_hbm = jnp.zeros((_SROWS, _H), jnp.float32)
    z1, r1 = _tc_prep(comp1, basis1, x, root1, bias1.reshape(1, _H))
    s1, cnt = _sc_msg_counts(z1.reshape(_SROWS, _H), src, dst, et, zeros_hbm)

    z2, r2, inv = _tc_mid(comp2, basis2, s1.reshape(_NC, _N, _R * _H),
                          cnt.reshape(_NC, _N, _R * _H), r1, root2,
                          bias2.reshape(1, _H))

    (s2,) = _sc_msg_plain(z2.reshape(_SROWS, _H), src, dst, et, zeros_hbm)

    (out,) = _tc_final(s2.reshape(_NC, _N, _R * _H), inv, r2,
                       fc1_W.T, fc1_b.reshape(1, _H))

    tot_src = jnp.concatenate([pos_edge_index[0], neg_edge_index[0]])
    tot_dst = jnp.concatenate([pos_edge_index[1], neg_edge_index[1]])
    (scores,) = _sc_scores(out, tot_src, tot_dst)
    return scores, out
